# baseline (speedup 1.0000x reference)
import jax
import jax.numpy as jnp
from jax.experimental import pallas as pl
from jax.experimental.pallas import tpu as pltpu

_B = 5000
_N = 20000
_C = 256
_CE = 384
_BT = 1280
_NT = 4096
_NB = 4
_NN = 5
_BP = _BT * _NB
_NP = _NT * _NN
_SCALE = 32768.0
_SHIFT = 1 << 24


def _roi_body(boxes_ref, coords_ref, feat_ref, mask_ref, bf_ref,
              fhe_ref, acc_ref):
    i_n = pl.program_id(0)
    i_b = pl.program_id(1)

    @pl.when(i_b == 0)
    def _stage_features():
        fhe_ref[:, :_C] = feat_ref[...].astype(jnp.bfloat16)
        fhe_ref[:, _C:] = jnp.concatenate(
            [jnp.ones((_NT, 1), jnp.bfloat16),
             jnp.zeros((_NT, _CE - _C - 1), jnp.bfloat16)], axis=1)

    @pl.when((i_b == 0) & (i_n == _NN - 1))
    def _zero_tail():
        fhe_ref[pl.ds(_N - (_NN - 1) * _NT, _NN * _NT - _N), :] = jnp.zeros(
            (_NN * _NT - _N, _CE), jnp.bfloat16)

    xq = coords_ref[0:1, :]
    yq = coords_ref[1:2, :]
    x0 = boxes_ref[:, 0:1]
    wx = boxes_ref[:, 1:2]
    y0 = boxes_ref[:, 2:3]
    wy = boxes_ref[:, 3:4]

    cx = (xq - x0) <= wx
    cy = (yq - y0) <= wy
    mi = jnp.where(cx, jnp.where(cy, jnp.int32(1), jnp.int32(0)),
                   jnp.int32(0))
    mask_ref[...] = mi.astype(jnp.int8)
    m = mi.astype(jnp.bfloat16)
    part = jnp.dot(m, fhe_ref[...], preferred_element_type=jnp.float32)

    rows = pl.ds(i_b * _BT, _BT)

    @pl.when(i_n == 0)
    def _init():
        acc_ref[rows, :] = part

    @pl.when(i_n > 0)
    def _acc():
        acc_ref[rows, :] = acc_ref[rows, :] + part

    @pl.when(i_n == _NN - 1)
    def _fin():
        tot = acc_ref[rows, :_C]
        c = jnp.maximum(acc_ref[rows, _C:_C + 1], 1.0)
        bf_ref[...] = tot / c


def kernel(coords, features, bbox_tensor, bbox_sample_association):
    f32, i32 = jnp.float32, jnp.int32

    xk = (coords[:, 0] * _SCALE).astype(i32)
    yk = (coords[:, 1] * _SCALE).astype(i32)
    sk = coords[:, 2].astype(i32)
    xq = xk + sk * _SHIFT
    a = bbox_sample_association.astype(i32)
    x0 = jnp.ceil(bbox_tensor[:, 0, 0] * _SCALE).astype(i32) + a * _SHIFT
    y0 = jnp.ceil(bbox_tensor[:, 0, 1] * _SCALE).astype(i32)
    x1 = (jnp.minimum(jnp.ceil(bbox_tensor[:, 1, 0] * _SCALE).astype(i32) - 1,
                      _SHIFT - 1) + a * _SHIFT)
    y1 = jnp.ceil(bbox_tensor[:, 1, 1] * _SCALE).astype(i32) - 1

    wx = x1 - x0
    wy = y1 - y0
    boxes_p = jnp.pad(jnp.stack([x0, wx, y0, wy], axis=-1),
                      ((0, _BP - _B), (0, 8 - 4)),
                      constant_values=0)
    boxes_p = boxes_p.at[_B:, 0].set(1 << 30)
    boxes_p = boxes_p.at[_B:, 2].set(1 << 30)
    boxes_p = boxes_p.astype(jnp.uint32)
    coords_q = jnp.pad(jnp.stack([xq, yk], axis=0),
                       ((0, 8 - 2), (0, _NP - _N)),
                       constant_values=-(1 << 31)).astype(jnp.uint32)

    grid = (_NN, _NB)
    is_inside, box_features = pl.pallas_call(
        _roi_body,
        grid=grid,
        in_specs=[
            pl.BlockSpec((_BT, 8), lambda i_n, i_b: (i_b, 0)),
            pl.BlockSpec((8, _NT), lambda i_n, i_b: (0, i_n)),
            pl.BlockSpec((_NT, _C), lambda i_n, i_b: (i_n, 0)),
        ],
        out_specs=[
            pl.BlockSpec((_BT, _NT), lambda i_n, i_b: (i_b, i_n)),
            pl.BlockSpec((_BT, _C), lambda i_n, i_b: (i_b, 0)),
        ],
        out_shape=[
            jax.ShapeDtypeStruct((_B, _N), jnp.int8),
            jax.ShapeDtypeStruct((_B, _C), jnp.float32),
        ],
        scratch_shapes=[
            pltpu.VMEM((_NT, _CE), jnp.bfloat16),
            pltpu.VMEM((_BP, _CE), jnp.float32),
        ],
    )(boxes_p, coords_q, features)
    return (box_features, is_inside.view(jnp.bool_))


# speedup vs baseline: 1.0302x; 1.0302x over previous
import jax
import jax.numpy as jnp
from jax.experimental import pallas as pl
from jax.experimental.pallas import tpu as pltpu

_B = 5000
_N = 20000
_C = 256
_CE = 384
_BT = 1024
_NT = 5120
_NB = 5
_NN = 4
_BP = _BT * _NB
_NP = _NT * _NN
_SCALE = 32768.0
_SHIFT = 1 << 24


def _roi_body(boxes_ref, coords_ref, feat_ref, mask_ref, bf_ref,
              fhe_ref, acc_ref):
    i_n = pl.program_id(0)
    i_b = pl.program_id(1)

    @pl.when(i_b == 0)
    def _stage_features():
        fhe_ref[:, :_C] = feat_ref[...].astype(jnp.bfloat16)
        fhe_ref[:, _C:] = jnp.concatenate(
            [jnp.ones((_NT, 1), jnp.bfloat16),
             jnp.zeros((_NT, _CE - _C - 1), jnp.bfloat16)], axis=1)

    @pl.when((i_b == 0) & (i_n == _NN - 1))
    def _zero_tail():
        fhe_ref[pl.ds(_N - (_NN - 1) * _NT, _NN * _NT - _N), :] = jnp.zeros(
            (_NN * _NT - _N, _CE), jnp.bfloat16)

    xq = coords_ref[0:1, :]
    yq = coords_ref[1:2, :]
    x0 = boxes_ref[:, 0:1]
    wx = boxes_ref[:, 1:2]
    y0 = boxes_ref[:, 2:3]
    wy = boxes_ref[:, 3:4]

    cx = (xq - x0) <= wx
    cy = (yq - y0) <= wy
    mi = jnp.where(cx, jnp.where(cy, jnp.int32(1), jnp.int32(0)),
                   jnp.int32(0))
    mask_ref[...] = mi.astype(jnp.int8)
    m = mi.astype(jnp.bfloat16)
    part = jnp.dot(m, fhe_ref[...], preferred_element_type=jnp.float32)

    rows = pl.ds(i_b * _BT, _BT)

    @pl.when(i_n == 0)
    def _init():
        acc_ref[rows, :] = part

    @pl.when(i_n > 0)
    def _acc():
        acc_ref[rows, :] = acc_ref[rows, :] + part

    @pl.when(i_n == _NN - 1)
    def _fin():
        tot = acc_ref[rows, :_C]
        c = jnp.maximum(acc_ref[rows, _C:_C + 1], 1.0)
        bf_ref[...] = tot / c


def kernel(coords, features, bbox_tensor, bbox_sample_association):
    f32, i32 = jnp.float32, jnp.int32

    xk = (coords[:, 0] * _SCALE).astype(i32)
    yk = (coords[:, 1] * _SCALE).astype(i32)
    sk = coords[:, 2].astype(i32)
    xq = xk + sk * _SHIFT
    a = bbox_sample_association.astype(i32)
    x0 = jnp.ceil(bbox_tensor[:, 0, 0] * _SCALE).astype(i32) + a * _SHIFT
    y0 = jnp.ceil(bbox_tensor[:, 0, 1] * _SCALE).astype(i32)
    x1 = (jnp.minimum(jnp.ceil(bbox_tensor[:, 1, 0] * _SCALE).astype(i32) - 1,
                      _SHIFT - 1) + a * _SHIFT)
    y1 = jnp.ceil(bbox_tensor[:, 1, 1] * _SCALE).astype(i32) - 1

    wx = x1 - x0
    wy = y1 - y0
    boxes_p = jnp.pad(jnp.stack([x0, wx, y0, wy], axis=-1),
                      ((0, _BP - _B), (0, 8 - 4)),
                      constant_values=0)
    boxes_p = boxes_p.at[_B:, 0].set(1 << 30)
    boxes_p = boxes_p.at[_B:, 2].set(1 << 30)
    boxes_p = boxes_p.astype(jnp.uint32)
    coords_q = jnp.pad(jnp.stack([xq, yk], axis=0),
                       ((0, 8 - 2), (0, _NP - _N)),
                       constant_values=-(1 << 31)).astype(jnp.uint32)

    grid = (_NN, _NB)
    is_inside, box_features = pl.pallas_call(
        _roi_body,
        grid=grid,
        in_specs=[
            pl.BlockSpec((_BT, 8), lambda i_n, i_b: (i_b, 0)),
            pl.BlockSpec((8, _NT), lambda i_n, i_b: (0, i_n)),
            pl.BlockSpec((_NT, _C), lambda i_n, i_b: (i_n, 0)),
        ],
        out_specs=[
            pl.BlockSpec((_BT, _NT), lambda i_n, i_b: (i_b, i_n)),
            pl.BlockSpec((_BT, _C), lambda i_n, i_b: (i_b, 0)),
        ],
        out_shape=[
            jax.ShapeDtypeStruct((_B, _N), jnp.int8),
            jax.ShapeDtypeStruct((_B, _C), jnp.float32),
        ],
        scratch_shapes=[
            pltpu.VMEM((_NT, _CE), jnp.bfloat16),
            pltpu.VMEM((_BP, _CE), jnp.float32),
        ],
    )(boxes_p, coords_q, features)
    return (box_features, is_inside.view(jnp.bool_))


# speedup vs baseline: 1.0439x; 1.0133x over previous
import jax
import jax.numpy as jnp
from jax.experimental import pallas as pl
from jax.experimental.pallas import tpu as pltpu

_B = 5000
_N = 20000
_C = 256
_CE = 384
_BT = 1024
_NT = 6912
_NB = 5
_NN = 3
_BP = _BT * _NB
_NP = _NT * _NN
_SCALE = 32768.0
_SHIFT = 1 << 24


def _roi_body(boxes_ref, coords_ref, feat_ref, mask_ref, bf_ref,
              fhe_ref, acc_ref):
    i_n = pl.program_id(0)
    i_b = pl.program_id(1)

    @pl.when(i_b == 0)
    def _stage_features():
        fhe_ref[:, :_C] = feat_ref[...].astype(jnp.bfloat16)
        fhe_ref[:, _C:] = jnp.concatenate(
            [jnp.ones((_NT, 1), jnp.bfloat16),
             jnp.zeros((_NT, _CE - _C - 1), jnp.bfloat16)], axis=1)

    @pl.when((i_b == 0) & (i_n == _NN - 1))
    def _zero_tail():
        fhe_ref[pl.ds(_N - (_NN - 1) * _NT, _NN * _NT - _N), :] = jnp.zeros(
            (_NN * _NT - _N, _CE), jnp.bfloat16)

    xq = coords_ref[0:1, :]
    yq = coords_ref[1:2, :]
    x0 = boxes_ref[:, 0:1]
    wx = boxes_ref[:, 1:2]
    y0 = boxes_ref[:, 2:3]
    wy = boxes_ref[:, 3:4]

    cx = (xq - x0) <= wx
    cy = (yq - y0) <= wy
    mi = jnp.where(cx, jnp.where(cy, jnp.int32(1), jnp.int32(0)),
                   jnp.int32(0))
    mask_ref[...] = mi.astype(jnp.int8)
    m = mi.astype(jnp.bfloat16)
    part = jnp.dot(m, fhe_ref[...], preferred_element_type=jnp.float32)

    rows = pl.ds(i_b * _BT, _BT)

    @pl.when(i_n == 0)
    def _init():
        acc_ref[rows, :] = part

    @pl.when(i_n > 0)
    def _acc():
        acc_ref[rows, :] = acc_ref[rows, :] + part

    @pl.when(i_n == _NN - 1)
    def _fin():
        tot = acc_ref[rows, :_C]
        c = jnp.maximum(acc_ref[rows, _C:_C + 1], 1.0)
        bf_ref[...] = tot / c


def kernel(coords, features, bbox_tensor, bbox_sample_association):
    f32, i32 = jnp.float32, jnp.int32

    xk = (coords[:, 0] * _SCALE).astype(i32)
    yk = (coords[:, 1] * _SCALE).astype(i32)
    sk = coords[:, 2].astype(i32)
    xq = xk + sk * _SHIFT
    a = bbox_sample_association.astype(i32)
    x0 = jnp.ceil(bbox_tensor[:, 0, 0] * _SCALE).astype(i32) + a * _SHIFT
    y0 = jnp.ceil(bbox_tensor[:, 0, 1] * _SCALE).astype(i32)
    x1 = (jnp.minimum(jnp.ceil(bbox_tensor[:, 1, 0] * _SCALE).astype(i32) - 1,
                      _SHIFT - 1) + a * _SHIFT)
    y1 = jnp.ceil(bbox_tensor[:, 1, 1] * _SCALE).astype(i32) - 1

    wx = x1 - x0
    wy = y1 - y0
    boxes_p = jnp.pad(jnp.stack([x0, wx, y0, wy], axis=-1),
                      ((0, _BP - _B), (0, 8 - 4)),
                      constant_values=0)
    boxes_p = boxes_p.at[_B:, 0].set(1 << 30)
    boxes_p = boxes_p.at[_B:, 2].set(1 << 30)
    boxes_p = boxes_p.astype(jnp.uint32)
    coords_q = jnp.pad(jnp.stack([xq, yk], axis=0),
                       ((0, 8 - 2), (0, _NP - _N)),
                       constant_values=-(1 << 31)).astype(jnp.uint32)

    grid = (_NN, _NB)
    is_inside, box_features = pl.pallas_call(
        _roi_body,
        grid=grid,
        in_specs=[
            pl.BlockSpec((_BT, 8), lambda i_n, i_b: (i_b, 0)),
            pl.BlockSpec((8, _NT), lambda i_n, i_b: (0, i_n)),
            pl.BlockSpec((_NT, _C), lambda i_n, i_b: (i_n, 0)),
        ],
        out_specs=[
            pl.BlockSpec((_BT, _NT), lambda i_n, i_b: (i_b, i_n)),
            pl.BlockSpec((_BT, _C), lambda i_n, i_b: (i_b, 0)),
        ],
        out_shape=[
            jax.ShapeDtypeStruct((_B, _N), jnp.int8),
            jax.ShapeDtypeStruct((_B, _C), jnp.float32),
        ],
        scratch_shapes=[
            pltpu.VMEM((_NT, _CE), jnp.bfloat16),
            pltpu.VMEM((_BP, _CE), jnp.float32),
        ],
    )(boxes_p, coords_q, features)
    return (box_features, is_inside.view(jnp.bool_))
